# baseline (speedup 1.0000x reference)
import functools

import jax
import jax.numpy as jnp
from jax import lax
from jax.experimental import pallas as pl
from jax.experimental.pallas import tpu as pltpu
from jax.experimental.pallas import tpu_sc as plsc

_N = 10000
_E = 320000
_D = 128
_R = 16

_BN = 2000
_NB = _N // _BN

_NC = 2
_NS = 16
_NW = _NC * _NS
_CH = 16
_NCH = 640
_NH = 80
_KB = 4
_EPW = _NCH * _CH
_EP = _NW * _EPW
_NP = 10240
_RPT = _NP // _NS


def _linz_body(x_ref, wwh_ref, b_ref, whr_ref, lin_ref, z_ref):
    lin = lax.dot_general(x_ref[...], wwh_ref[...], (((1,), (1,)), ((), ())),
                          preferred_element_type=jnp.float32) + b_ref[...]
    lin_ref[...] = lin
    z_ref[...] = lax.dot_general(jnp.tanh(lin), whr_ref[...],
                                 (((1,), (1,)), ((), ())),
                                 preferred_element_type=jnp.float32)


def _y_body(z_ref, ws_ref, y_ref):
    y_ref[0] = lax.dot_general(z_ref[...], ws_ref[0], (((1,), (1,)), ((), ())),
                               preferred_element_type=jnp.float32)


def _idx_body(t_ref, s_ref, o_ref):
    o_ref[...] = t_ref[...] * _N + s_ref[...]


def _final_body(lin_ref, p_ref, h_ref):
    h_ref[...] = jnp.tanh(lin_ref[...] + p_ref[0] + p_ref[1])


def _sc_agg(y2, flat3, dst3, zeros):
    mesh = plsc.VectorSubcoreMesh(core_axis_name="c", subcore_axis_name="s")

    @functools.partial(
        pl.kernel,
        out_type=jax.ShapeDtypeStruct((_NC, _NP, _D), jnp.float32),
        mesh=mesh,
        scratch_types=[
            pltpu.VMEM((_NH, _CH), jnp.int32),
            pltpu.VMEM((_NH, _CH), jnp.int32),
            [pltpu.VMEM((_CH, _D), jnp.float32) for _ in range(_KB)],
            pltpu.VMEM_SHARED((_NP, _D), jnp.float32),
            [pltpu.SemaphoreType.DMA for _ in range(_KB)],
        ],
    )
    def k(y_hbm, flat_hbm, dst_hbm, zero_hbm, out_hbm,
          idx_v, dst_v, rows, acc, sems):
        c = lax.axis_index("c")
        s = lax.axis_index("s")
        wid = s * _NC + c
        pltpu.sync_copy(zero_hbm.at[pl.ds(s * _RPT, _RPT)],
                        acc.at[pl.ds(s * _RPT, _RPT)])
        plsc.subcore_barrier()

        for phase in range(_NCH // _NH):
            pltpu.sync_copy(flat_hbm.at[wid, pl.ds(phase * _NH, _NH)], idx_v)
            pltpu.sync_copy(dst_hbm.at[wid, pl.ds(phase * _NH, _NH)], dst_v)
            for b in range(_KB):
                pltpu.async_copy(y_hbm.at[idx_v.at[b]], rows[b], sems[b])

            def body(q, carry):
                for b in range(_KB):
                    g = _KB * q + b
                    pltpu.make_async_copy(y_hbm.at[idx_v.at[g]], rows[b],
                                          sems[b]).wait()
                    pltpu.sync_copy(rows[b], acc.at[dst_v.at[g]], add=True)
                    nxt = jnp.minimum(g + _KB, _NH - 1)
                    pltpu.async_copy(y_hbm.at[idx_v.at[nxt]], rows[b],
                                     sems[b])
                return carry

            lax.fori_loop(0, _NH // _KB, body, 0)
            for b in range(_KB):
                pltpu.make_async_copy(y_hbm.at[idx_v.at[_NH - 1]], rows[b],
                                      sems[b]).wait()
        plsc.subcore_barrier()
        pltpu.sync_copy(acc.at[pl.ds(s * _RPT, _RPT)],
                        out_hbm.at[c, pl.ds(s * _RPT, _RPT)])

    return k(y2, flat3, dst3, zeros)


def kernel(x, edge_index, edge_type, w_wordvec_hidden, b_wordvec_hidden,
           w_hidden_relation, ws_relation_hidden):
    b2 = b_wordvec_hidden.reshape(1, _D)

    lin, z = pl.pallas_call(
        _linz_body,
        grid=(_NB,),
        in_specs=[
            pl.BlockSpec((_BN, _D), lambda i: (i, 0)),
            pl.BlockSpec((_D, _D), lambda i: (0, 0)),
            pl.BlockSpec((1, _D), lambda i: (0, 0)),
            pl.BlockSpec((_D, _D), lambda i: (0, 0)),
        ],
        out_specs=[
            pl.BlockSpec((_BN, _D), lambda i: (i, 0)),
            pl.BlockSpec((_BN, _D), lambda i: (i, 0)),
        ],
        out_shape=[
            jax.ShapeDtypeStruct((_N, _D), jnp.float32),
            jax.ShapeDtypeStruct((_N, _D), jnp.float32),
        ],
    )(x, w_wordvec_hidden, b2, w_hidden_relation)

    y = pl.pallas_call(
        _y_body,
        grid=(_NB, _R),
        in_specs=[
            pl.BlockSpec((_BN, _D), lambda i, r: (i, 0)),
            pl.BlockSpec((1, _D, _D), lambda i, r: (r, 0, 0)),
        ],
        out_specs=pl.BlockSpec((1, _BN, _D), lambda i, r: (r, i, 0)),
        out_shape=jax.ShapeDtypeStruct((_R, _N, _D), jnp.float32),
    )(z, ws_relation_hidden)
    y2 = y.reshape(_R * _N, _D)

    pad = _EP - _E
    zpad = jnp.zeros((pad,), jnp.int32)
    t2 = jnp.concatenate([edge_type, zpad]).reshape(_EP // _D, _D)
    s2 = jnp.concatenate([edge_index[0], zpad]).reshape(_EP // _D, _D)
    dst3 = jnp.concatenate(
        [edge_index[1], jnp.full((pad,), _N, jnp.int32)]
    ).reshape(_NW, _NCH, _CH)

    flat = pl.pallas_call(
        _idx_body,
        grid=(1,),
        in_specs=[
            pl.BlockSpec((_EP // _D, _D), lambda i: (0, 0)),
            pl.BlockSpec((_EP // _D, _D), lambda i: (0, 0)),
        ],
        out_specs=pl.BlockSpec((_EP // _D, _D), lambda i: (0, 0)),
        out_shape=jax.ShapeDtypeStruct((_EP // _D, _D), jnp.int32),
    )(t2, s2)
    flat3 = flat.reshape(_NW, _NCH, _CH)
    zeros = jnp.zeros((_NP, _D), jnp.float32)

    parts = _sc_agg(y2, flat3, dst3, zeros)

    h = pl.pallas_call(
        _final_body,
        grid=(_NB,),
        in_specs=[
            pl.BlockSpec((_BN, _D), lambda i: (i, 0)),
            pl.BlockSpec((_NC, _BN, _D), lambda i: (0, i, 0)),
        ],
        out_specs=pl.BlockSpec((_BN, _D), lambda i: (i, 0)),
        out_shape=jax.ShapeDtypeStruct((_N, _D), jnp.float32),
    )(lin, parts)
    return h


# speedup vs baseline: 1.0747x; 1.0747x over previous
import functools

import jax
import jax.numpy as jnp
from jax import lax
from jax.experimental import pallas as pl
from jax.experimental.pallas import tpu as pltpu
from jax.experimental.pallas import tpu_sc as plsc

_N = 10000
_E = 320000
_D = 128
_R = 16

_BN = 2000
_NB = _N // _BN

_NC = 2
_NS = 16
_NW = _NC * _NS
_CH = 32
_NCH = 320
_NH = 80
_KB = 4
_EPW = _NCH * _CH
_EP = _NW * _EPW
_NP = 10240
_RPT = _NP // _NS


def _linz_body(x_ref, wwh_ref, b_ref, whr_ref, lin_ref, z_ref):
    lin = lax.dot_general(x_ref[...], wwh_ref[...], (((1,), (1,)), ((), ())),
                          preferred_element_type=jnp.float32) + b_ref[...]
    lin_ref[...] = lin
    z_ref[...] = lax.dot_general(jnp.tanh(lin), whr_ref[...],
                                 (((1,), (1,)), ((), ())),
                                 preferred_element_type=jnp.float32)


def _y_body(z_ref, ws_ref, y_ref):
    y_ref[0] = lax.dot_general(z_ref[...], ws_ref[0], (((1,), (1,)), ((), ())),
                               preferred_element_type=jnp.float32)


def _idx_body(t_ref, s_ref, o_ref):
    o_ref[...] = t_ref[...] * _N + s_ref[...]


def _final_body(lin_ref, p_ref, h_ref):
    h_ref[...] = jnp.tanh(lin_ref[...] + p_ref[0] + p_ref[1])


def _sc_agg(y2, flat3, dst3, zeros):
    mesh = plsc.VectorSubcoreMesh(core_axis_name="c", subcore_axis_name="s")

    @functools.partial(
        pl.kernel,
        out_type=jax.ShapeDtypeStruct((_NC, _NP, _D), jnp.float32),
        mesh=mesh,
        scratch_types=[
            pltpu.VMEM((_NH, _CH), jnp.int32),
            pltpu.VMEM((_NH, _CH), jnp.int32),
            [pltpu.VMEM((_CH, _D), jnp.float32) for _ in range(_KB)],
            pltpu.VMEM_SHARED((_NP, _D), jnp.float32),
            [pltpu.SemaphoreType.DMA for _ in range(_KB)],
        ],
    )
    def k(y_hbm, flat_hbm, dst_hbm, zero_hbm, out_hbm,
          idx_v, dst_v, rows, acc, sems):
        c = lax.axis_index("c")
        s = lax.axis_index("s")
        wid = s * _NC + c
        pltpu.sync_copy(zero_hbm.at[pl.ds(s * _RPT, _RPT)],
                        acc.at[pl.ds(s * _RPT, _RPT)])
        plsc.subcore_barrier()

        for phase in range(_NCH // _NH):
            pltpu.sync_copy(flat_hbm.at[wid, pl.ds(phase * _NH, _NH)], idx_v)
            pltpu.sync_copy(dst_hbm.at[wid, pl.ds(phase * _NH, _NH)], dst_v)
            for b in range(_KB):
                pltpu.async_copy(y_hbm.at[idx_v.at[b]], rows[b], sems[b])

            def body(q, carry):
                for b in range(_KB):
                    g = _KB * q + b
                    pltpu.make_async_copy(y_hbm.at[idx_v.at[g]], rows[b],
                                          sems[b]).wait()
                    pltpu.sync_copy(rows[b], acc.at[dst_v.at[g]], add=True)
                    nxt = jnp.minimum(g + _KB, _NH - 1)
                    pltpu.async_copy(y_hbm.at[idx_v.at[nxt]], rows[b],
                                     sems[b])
                return carry

            lax.fori_loop(0, _NH // _KB, body, 0)
            for b in range(_KB):
                pltpu.make_async_copy(y_hbm.at[idx_v.at[_NH - 1]], rows[b],
                                      sems[b]).wait()
        plsc.subcore_barrier()
        pltpu.sync_copy(acc.at[pl.ds(s * _RPT, _RPT)],
                        out_hbm.at[c, pl.ds(s * _RPT, _RPT)])

    return k(y2, flat3, dst3, zeros)


def kernel(x, edge_index, edge_type, w_wordvec_hidden, b_wordvec_hidden,
           w_hidden_relation, ws_relation_hidden):
    b2 = b_wordvec_hidden.reshape(1, _D)

    lin, z = pl.pallas_call(
        _linz_body,
        grid=(_NB,),
        in_specs=[
            pl.BlockSpec((_BN, _D), lambda i: (i, 0)),
            pl.BlockSpec((_D, _D), lambda i: (0, 0)),
            pl.BlockSpec((1, _D), lambda i: (0, 0)),
            pl.BlockSpec((_D, _D), lambda i: (0, 0)),
        ],
        out_specs=[
            pl.BlockSpec((_BN, _D), lambda i: (i, 0)),
            pl.BlockSpec((_BN, _D), lambda i: (i, 0)),
        ],
        out_shape=[
            jax.ShapeDtypeStruct((_N, _D), jnp.float32),
            jax.ShapeDtypeStruct((_N, _D), jnp.float32),
        ],
    )(x, w_wordvec_hidden, b2, w_hidden_relation)

    y = pl.pallas_call(
        _y_body,
        grid=(_NB, _R),
        in_specs=[
            pl.BlockSpec((_BN, _D), lambda i, r: (i, 0)),
            pl.BlockSpec((1, _D, _D), lambda i, r: (r, 0, 0)),
        ],
        out_specs=pl.BlockSpec((1, _BN, _D), lambda i, r: (r, i, 0)),
        out_shape=jax.ShapeDtypeStruct((_R, _N, _D), jnp.float32),
    )(z, ws_relation_hidden)
    y2 = y.reshape(_R * _N, _D)

    pad = _EP - _E
    zpad = jnp.zeros((pad,), jnp.int32)
    t2 = jnp.concatenate([edge_type, zpad]).reshape(_EP // _D, _D)
    s2 = jnp.concatenate([edge_index[0], zpad]).reshape(_EP // _D, _D)
    dst3 = jnp.concatenate(
        [edge_index[1], jnp.full((pad,), _N, jnp.int32)]
    ).reshape(_NW, _NCH, _CH)

    flat = pl.pallas_call(
        _idx_body,
        grid=(1,),
        in_specs=[
            pl.BlockSpec((_EP // _D, _D), lambda i: (0, 0)),
            pl.BlockSpec((_EP // _D, _D), lambda i: (0, 0)),
        ],
        out_specs=pl.BlockSpec((_EP // _D, _D), lambda i: (0, 0)),
        out_shape=jax.ShapeDtypeStruct((_EP // _D, _D), jnp.int32),
    )(t2, s2)
    flat3 = flat.reshape(_NW, _NCH, _CH)
    zeros = jnp.zeros((_NP, _D), jnp.float32)

    parts = _sc_agg(y2, flat3, dst3, zeros)

    h = pl.pallas_call(
        _final_body,
        grid=(_NB,),
        in_specs=[
            pl.BlockSpec((_BN, _D), lambda i: (i, 0)),
            pl.BlockSpec((_NC, _BN, _D), lambda i: (0, i, 0)),
        ],
        out_specs=pl.BlockSpec((_BN, _D), lambda i: (i, 0)),
        out_shape=jax.ShapeDtypeStruct((_N, _D), jnp.float32),
    )(lin, parts)
    return h
